# baseline (speedup 1.0000x reference)
import functools

import jax
import jax.numpy as jnp
from jax import lax
from jax.experimental import pallas as pl
from jax.experimental.pallas import tpu as pltpu
from jax.experimental.pallas import tpu_sc as plsc

B, S, D_IN, D_LAT, K = 64, 1024, 256, 64, 1024
N = B * S

T = 512
G = N // T

NUM_CORES = 2
NUM_SUBCORES = 16
NW = NUM_CORES * NUM_SUBCORES
PER_W = N // NW
CHUNK = 128
NCHUNK = PER_W // CHUNK


def _tc_body(x_ref, w_enc_ref, b_enc_ref, cb_ref, z_e_ref, idx_ref):
    x = x_ref[...]
    w = w_enc_ref[...]
    b = b_enc_ref[...]
    cb = cb_ref[...]
    z_e = jnp.dot(x, w, preferred_element_type=jnp.float32) + b
    z_e_ref[...] = z_e
    scores = lax.dot_general(z_e, cb, (((1,), (1,)), ((), ())),
                             preferred_element_type=jnp.float32)
    z_sq = jnp.sum(z_e * z_e, axis=1, keepdims=True)
    cb_sq = lax.dot_general(jnp.ones((1, D_LAT), jnp.float32), cb * cb,
                            (((1,), (1,)), ((), ())),
                            precision=lax.Precision.HIGHEST,
                            preferred_element_type=jnp.float32)
    d = (z_sq - 2.0 * scores) + cb_sq
    m = jnp.min(d, axis=1, keepdims=True)
    iota = lax.broadcasted_iota(jnp.int32, (T, K), 1)
    idx = jnp.min(jnp.where(d == m, iota, K), axis=1, keepdims=True)
    idx_ref[...] = idx


def _cdec_body(cb_ref, w_dec_ref, b_dec_ref, out_ref):
    out_ref[...] = (jnp.dot(cb_ref[...], w_dec_ref[...],
                            preferred_element_type=jnp.float32)
                    + b_dec_ref[...])


@functools.cache
def _make_sc_gather():
    mesh = plsc.VectorSubcoreMesh(core_axis_name="c", subcore_axis_name="s")

    @functools.partial(
        pl.kernel,
        mesh=mesh,
        out_type=[
            jax.ShapeDtypeStruct((N, 128), jnp.float32),
            jax.ShapeDtypeStruct((N, D_IN), jnp.float32),
        ],
        scratch_types=[
            pltpu.VMEM((CHUNK,), jnp.int32),
            pltpu.VMEM((CHUNK, 128), jnp.float32),
            pltpu.VMEM((CHUNK, D_IN), jnp.float32),
            pltpu.SemaphoreType.DMA,
            pltpu.SemaphoreType.DMA,
        ],
    )
    def _sc_gather(cb_hbm, cdec_hbm, idx_hbm, zq_hbm, xrec_hbm,
                   idx_v, zq_v, xr_v, sem1, sem2):
        wid = lax.axis_index("s") * NUM_CORES + lax.axis_index("c")
        base = wid * PER_W

        def body(j, carry):
            off = base + j * CHUNK
            pltpu.sync_copy(idx_hbm.at[pl.ds(off, CHUNK)], idx_v)
            g1 = pltpu.async_copy(cb_hbm.at[idx_v], zq_v, sem1)
            g2 = pltpu.async_copy(cdec_hbm.at[idx_v], xr_v, sem2)
            g1.wait()
            g2.wait()
            pltpu.sync_copy(zq_v, zq_hbm.at[pl.ds(off, CHUNK)])
            pltpu.sync_copy(xr_v, xrec_hbm.at[pl.ds(off, CHUNK)])
            return carry

        lax.fori_loop(0, NCHUNK, body, 0)

    return _sc_gather


def kernel(x, W_enc, b_enc, W_dec, b_dec, codebook):
    x2 = x.reshape(N, D_IN)
    b_enc2 = b_enc.reshape(1, D_LAT)

    z_e_flat, idx2 = pl.pallas_call(
        _tc_body,
        grid=(G,),
        in_specs=[
            pl.BlockSpec((T, D_IN), lambda i: (i, 0)),
            pl.BlockSpec((D_IN, D_LAT), lambda i: (0, 0)),
            pl.BlockSpec((1, D_LAT), lambda i: (0, 0)),
            pl.BlockSpec((K, D_LAT), lambda i: (0, 0)),
        ],
        out_specs=[
            pl.BlockSpec((T, D_LAT), lambda i: (i, 0)),
            pl.BlockSpec((T, 1), lambda i: (i, 0)),
        ],
        out_shape=[
            jax.ShapeDtypeStruct((N, D_LAT), jnp.float32),
            jax.ShapeDtypeStruct((N, 1), jnp.int32),
        ],
    )(x2, W_enc, b_enc2, codebook)

    c_dec = pl.pallas_call(
        _cdec_body,
        out_shape=jax.ShapeDtypeStruct((K, D_IN), jnp.float32),
    )(codebook, W_dec, b_dec.reshape(1, D_IN))

    idx_flat = idx2.reshape(N)
    cb_pad = jnp.pad(codebook, ((0, 0), (0, 128 - D_LAT)))
    z_q_pad, x_rec_flat = _make_sc_gather()(cb_pad, c_dec, idx_flat)
    z_q_flat = z_q_pad[:, :D_LAT]

    return (z_e_flat.reshape(B, S, D_LAT),
            z_q_flat.reshape(B, S, D_LAT),
            x_rec_flat.reshape(B, S, D_IN),
            idx2.reshape(B, S))
